# initial kernel scaffold (unmeasured)
import jax
import jax.numpy as jnp
from jax import lax
from jax.experimental import pallas as pl
from jax.experimental.pallas import tpu as pltpu

S = 1024
D = 2048
DC = 128
H = 16
DH = 128
DR = 32
SCALE = (DH + DR) ** -0.5

F32 = jnp.float32


def kernel(x, Wdkv, Wuk, Wuv, Wq, Wqr, Wkr, Wo):
    x2d = x.reshape(S, D)

    def body(x_ref, wdkv_ref, wuk_ref, wuv_ref, wq_ref, wqr_ref, wkr_ref,
             wo_ref, out_ref,
             c_send, c_recv, wuk_recv, wuv_recv,
             k_buf, v_buf, qr_buf, kr_buf, o_buf,
             send_sems, recv_sems):
        my_x = lax.axis_index("x")
        my_y = lax.axis_index("y")
        my_z = lax.axis_index("z")
        peer = (1 - my_x, my_y, my_z)

        barrier_sem = pltpu.get_barrier_semaphore()
        pl.semaphore_signal(barrier_sem, inc=1, device_id=peer,
                            device_id_type=pl.DeviceIdType.MESH)
        pl.semaphore_wait(barrier_sem, 1)

        xv = x_ref[...]

        c_send[...] = jnp.dot(xv, wdkv_ref[...], preferred_element_type=F32)

        rdma_c = pltpu.make_async_remote_copy(
            src_ref=c_send, dst_ref=c_recv,
            send_sem=send_sems.at[0], recv_sem=recv_sems.at[0],
            device_id=peer, device_id_type=pl.DeviceIdType.MESH)
        rdma_c.start()
        rdma_wk = pltpu.make_async_remote_copy(
            src_ref=wuk_ref, dst_ref=wuk_recv,
            send_sem=send_sems.at[1], recv_sem=recv_sems.at[1],
            device_id=peer, device_id_type=pl.DeviceIdType.MESH)
        rdma_wk.start()
        rdma_wv = pltpu.make_async_remote_copy(
            src_ref=wuv_ref, dst_ref=wuv_recv,
            send_sem=send_sems.at[2], recv_sem=recv_sems.at[2],
            device_id=peer, device_id_type=pl.DeviceIdType.MESH)
        rdma_wv.start()

        qr_buf[...] = jnp.dot(xv, wqr_ref[...], preferred_element_type=F32)
        kr_buf[...] = jnp.dot(xv, wkr_ref[...], preferred_element_type=F32)

        rdma_c.wait()
        rdma_wk.wait()
        rdma_wv.wait()

        c_mine = c_send[...]
        c_peer = c_recv[...]
        k_buf[...] = (jnp.dot(c_mine, wuk_ref[...], preferred_element_type=F32)
                      + jnp.dot(c_peer, wuk_recv[...], preferred_element_type=F32))
        v_buf[...] = (jnp.dot(c_mine, wuv_ref[...], preferred_element_type=F32)
                      + jnp.dot(c_peer, wuv_recv[...], preferred_element_type=F32))

        kr = kr_buf[...]

        for h in range(H):
            q_h = jnp.dot(xv, wq_ref[:, h * DH:(h + 1) * DH],
                          preferred_element_type=F32)
            k_h = k_buf[:, h * DH:(h + 1) * DH]
            qr_h = qr_buf[:, h * DR:(h + 1) * DR]
            s = lax.dot_general(q_h, k_h, (((1,), (1,)), ((), ())),
                                preferred_element_type=F32)
            s = s + lax.dot_general(qr_h, kr, (((1,), (1,)), ((), ())),
                                    preferred_element_type=F32)
            s = s * SCALE
            m = jnp.max(s, axis=1, keepdims=True)
            p = jnp.exp(s - m)
            p = p / jnp.sum(p, axis=1, keepdims=True)
            o_buf[:, h * DH:(h + 1) * DH] = jnp.dot(
                p, v_buf[:, h * DH:(h + 1) * DH], preferred_element_type=F32)

        out_ref[...] = jnp.dot(o_buf[...], wo_ref[...],
                               preferred_element_type=F32)

    out = pl.pallas_call(
        body,
        out_shape=jax.ShapeDtypeStruct((S, D), F32),
        in_specs=[pl.BlockSpec(memory_space=pltpu.VMEM)] * 8,
        out_specs=pl.BlockSpec(memory_space=pltpu.VMEM),
        scratch_shapes=[
            pltpu.VMEM((S, DC), F32),
            pltpu.VMEM((S, DC), F32),
            pltpu.VMEM((DC, D), F32),
            pltpu.VMEM((DC, D), F32),
            pltpu.VMEM((S, D), F32),
            pltpu.VMEM((S, D), F32),
            pltpu.VMEM((S, H * DR), F32),
            pltpu.VMEM((S, DR), F32),
            pltpu.VMEM((S, D), F32),
            pltpu.SemaphoreType.DMA((3,)),
            pltpu.SemaphoreType.DMA((3,)),
        ],
        compiler_params=pltpu.CompilerParams(
            collective_id=0,
            vmem_limit_bytes=128 * 1024 * 1024,
        ),
    )(x2d, Wdkv, Wuk, Wuv, Wq, Wqr, Wkr, Wo)
    return out.reshape(1, S, D)


# baseline (device time: 192000 ns/iter reference)
import jax
import jax.numpy as jnp
from jax import lax
from jax.experimental import pallas as pl
from jax.experimental.pallas import tpu as pltpu

S = 1024
D = 2048
DC = 128
H = 16
DH = 128
DR = 32
SCALE = (DH + DR) ** -0.5

F32 = jnp.float32


def kernel(x, Wdkv, Wuk, Wuv, Wq, Wqr, Wkr, Wo):
    x2d = x.reshape(S, D)
    wqr3 = Wqr.reshape(D, H, DR).transpose(1, 0, 2)

    def body(x_ref, wdkv_ref, wuk_ref, wuv_ref, wq_ref, wqr_ref, wkr_ref,
             wo_ref, out_ref,
             c_send, c_recv, wuk_recv, wuv_recv, kr_buf,
             send_sems, recv_sems):
        h = pl.program_id(0)
        my_x = lax.axis_index("x")
        my_y = lax.axis_index("y")
        my_z = lax.axis_index("z")
        peer = (1 - my_x, my_y, my_z)

        @pl.when(h == 0)
        def _exchange():
            barrier_sem = pltpu.get_barrier_semaphore()
            pl.semaphore_signal(barrier_sem, inc=1, device_id=peer,
                                device_id_type=pl.DeviceIdType.MESH)
            pl.semaphore_wait(barrier_sem, 1)

            c_send[...] = jnp.dot(x_ref[...], wdkv_ref[...],
                                  preferred_element_type=F32)

            rdma_c = pltpu.make_async_remote_copy(
                src_ref=c_send, dst_ref=c_recv,
                send_sem=send_sems.at[0], recv_sem=recv_sems.at[0],
                device_id=peer, device_id_type=pl.DeviceIdType.MESH)
            rdma_c.start()
            rdma_wk = pltpu.make_async_remote_copy(
                src_ref=wuk_ref, dst_ref=wuk_recv,
                send_sem=send_sems.at[1], recv_sem=recv_sems.at[1],
                device_id=peer, device_id_type=pl.DeviceIdType.MESH)
            rdma_wk.start()
            rdma_wv = pltpu.make_async_remote_copy(
                src_ref=wuv_ref, dst_ref=wuv_recv,
                send_sem=send_sems.at[2], recv_sem=recv_sems.at[2],
                device_id=peer, device_id_type=pl.DeviceIdType.MESH)
            rdma_wv.start()

            kr_buf[...] = jnp.dot(x_ref[...], wkr_ref[...],
                                  preferred_element_type=F32)

            rdma_c.wait()
            rdma_wk.wait()
            rdma_wv.wait()

        xv = x_ref[...]
        q_h = jnp.dot(xv, wq_ref[...], preferred_element_type=F32)
        qr_h = jnp.dot(xv, wqr_ref[0], preferred_element_type=F32)

        c_m = c_send[...]
        c_p = c_recv[...]
        cols = pl.ds(h * DH, DH)
        k_h = (jnp.dot(c_m, wuk_ref[:, cols], preferred_element_type=F32)
               + jnp.dot(c_p, wuk_recv[:, cols], preferred_element_type=F32))
        v_h = (jnp.dot(c_m, wuv_ref[:, cols], preferred_element_type=F32)
               + jnp.dot(c_p, wuv_recv[:, cols], preferred_element_type=F32))

        s = lax.dot_general(q_h, k_h, (((1,), (1,)), ((), ())),
                            preferred_element_type=F32)
        s = s + lax.dot_general(qr_h, kr_buf[...], (((1,), (1,)), ((), ())),
                                preferred_element_type=F32)
        s = s * SCALE
        m = jnp.max(s, axis=1, keepdims=True)
        p = jnp.exp(s - m)
        p = p / jnp.sum(p, axis=1, keepdims=True)
        o_h = jnp.dot(p, v_h, preferred_element_type=F32)
        contrib = jnp.dot(o_h, wo_ref[...], preferred_element_type=F32)

        @pl.when(h == 0)
        def _init():
            out_ref[...] = contrib

        @pl.when(h != 0)
        def _acc():
            out_ref[...] = out_ref[...] + contrib

    out = pl.pallas_call(
        body,
        grid=(H,),
        out_shape=jax.ShapeDtypeStruct((S, D), F32),
        in_specs=[
            pl.BlockSpec((S, D), lambda h: (0, 0)),
            pl.BlockSpec((D, DC), lambda h: (0, 0)),
            pl.BlockSpec((DC, D), lambda h: (0, 0)),
            pl.BlockSpec((DC, D), lambda h: (0, 0)),
            pl.BlockSpec((D, DH), lambda h: (0, h)),
            pl.BlockSpec((1, D, DR), lambda h: (h, 0, 0)),
            pl.BlockSpec((D, DR), lambda h: (0, 0)),
            pl.BlockSpec((DH, D), lambda h: (h, 0)),
        ],
        out_specs=pl.BlockSpec((S, D), lambda h: (0, 0)),
        scratch_shapes=[
            pltpu.VMEM((S, DC), F32),
            pltpu.VMEM((S, DC), F32),
            pltpu.VMEM((DC, D), F32),
            pltpu.VMEM((DC, D), F32),
            pltpu.VMEM((S, DR), F32),
            pltpu.SemaphoreType.DMA((3,)),
            pltpu.SemaphoreType.DMA((3,)),
        ],
        compiler_params=pltpu.CompilerParams(
            collective_id=0,
            vmem_limit_bytes=64 * 1024 * 1024,
        ),
    )(x2d, Wdkv, Wuk, Wuv, Wq, wqr3, Wkr, Wo)
    return out.reshape(1, S, D)


# device time: 133843 ns/iter; 1.4345x vs baseline; 1.4345x over previous
import jax
import jax.numpy as jnp
from jax import lax
from jax.experimental import pallas as pl
from jax.experimental.pallas import tpu as pltpu

S = 1024
D = 2048
DC = 128
H = 16
G = 4
DH = 128
DR = 32
SCALE = (DH + DR) ** -0.5

F32 = jnp.float32


def kernel(x, Wdkv, Wuk, Wuv, Wq, Wqr, Wkr, Wo):
    x2d = x.reshape(S, D)

    def body(x_ref, wdkv_ref, wuk_ref, wuv_ref, wq_ref, wqr_ref, wkr_ref,
             wo_ref, out_ref,
             c_send, c_recv, wuk_recv, wuv_recv, kr_buf,
             send_sems, recv_sems):
        h = pl.program_id(0)
        my_x = lax.axis_index("x")
        my_y = lax.axis_index("y")
        my_z = lax.axis_index("z")
        peer = (1 - my_x, my_y, my_z)

        def mk_rdma(i, src, dst):
            return pltpu.make_async_remote_copy(
                src_ref=src, dst_ref=dst,
                send_sem=send_sems.at[i], recv_sem=recv_sems.at[i],
                device_id=peer, device_id_type=pl.DeviceIdType.MESH)

        @pl.when(h == 0)
        def _exchange():
            barrier_sem = pltpu.get_barrier_semaphore()
            pl.semaphore_signal(barrier_sem, inc=1, device_id=peer,
                                device_id_type=pl.DeviceIdType.MESH)
            pl.semaphore_wait(barrier_sem, 1)

            c_send[...] = jnp.dot(x_ref[...], wdkv_ref[...],
                                  preferred_element_type=F32)

            mk_rdma(0, c_send, c_recv).start()
            mk_rdma(1, wuk_ref, wuk_recv).start()
            mk_rdma(2, wuv_ref, wuv_recv).start()

            kr_buf[...] = jnp.dot(x_ref[...], wkr_ref[...],
                                  preferred_element_type=F32)

        xv = x_ref[...]
        q_blk = jnp.dot(xv, wq_ref[...], preferred_element_type=F32)
        qr_blk = jnp.dot(xv, wqr_ref[...], preferred_element_type=F32)

        @pl.when(h == 0)
        def _wait_exchange():
            mk_rdma(0, c_send, c_recv).wait()
            mk_rdma(1, wuk_ref, wuk_recv).wait()
            mk_rdma(2, wuv_ref, wuv_recv).wait()

        c_m = c_send[...]
        c_p = c_recv[...]
        kr = kr_buf[...]

        o_parts = []
        for j in range(G):
            cols = pl.ds(h * (G * DH) + j * DH, DH)
            k_h = (jnp.dot(c_m, wuk_ref[:, cols], preferred_element_type=F32)
                   + jnp.dot(c_p, wuk_recv[:, cols],
                             preferred_element_type=F32))
            v_h = (jnp.dot(c_m, wuv_ref[:, cols], preferred_element_type=F32)
                   + jnp.dot(c_p, wuv_recv[:, cols],
                             preferred_element_type=F32))
            q_h = q_blk[:, j * DH:(j + 1) * DH]
            qr_h = qr_blk[:, j * DR:(j + 1) * DR]
            s = lax.dot_general(q_h, k_h, (((1,), (1,)), ((), ())),
                                preferred_element_type=F32)
            s = s + lax.dot_general(qr_h, kr, (((1,), (1,)), ((), ())),
                                    preferred_element_type=F32)
            s = s * SCALE
            m = jnp.max(s, axis=1, keepdims=True)
            p = jnp.exp(s - m)
            p = p / jnp.sum(p, axis=1, keepdims=True)
            o_parts.append(jnp.dot(p, v_h, preferred_element_type=F32))

        o_blk = jnp.concatenate(o_parts, axis=1)
        contrib = jnp.dot(o_blk, wo_ref[...], preferred_element_type=F32)

        @pl.when(h == 0)
        def _init():
            out_ref[...] = contrib

        @pl.when(h != 0)
        def _acc():
            out_ref[...] = out_ref[...] + contrib

    out = pl.pallas_call(
        body,
        grid=(H // G,),
        out_shape=jax.ShapeDtypeStruct((S, D), F32),
        in_specs=[
            pl.BlockSpec((S, D), lambda h: (0, 0)),
            pl.BlockSpec((D, DC), lambda h: (0, 0)),
            pl.BlockSpec((DC, D), lambda h: (0, 0)),
            pl.BlockSpec((DC, D), lambda h: (0, 0)),
            pl.BlockSpec((D, G * DH), lambda h: (0, h)),
            pl.BlockSpec((D, G * DR), lambda h: (0, h)),
            pl.BlockSpec((D, DR), lambda h: (0, 0)),
            pl.BlockSpec((G * DH, D), lambda h: (h, 0)),
        ],
        out_specs=pl.BlockSpec((S, D), lambda h: (0, 0)),
        scratch_shapes=[
            pltpu.VMEM((S, DC), F32),
            pltpu.VMEM((S, DC), F32),
            pltpu.VMEM((DC, D), F32),
            pltpu.VMEM((DC, D), F32),
            pltpu.VMEM((S, DR), F32),
            pltpu.SemaphoreType.DMA((3,)),
            pltpu.SemaphoreType.DMA((3,)),
        ],
        compiler_params=pltpu.CompilerParams(
            collective_id=0,
            vmem_limit_bytes=64 * 1024 * 1024,
        ),
    )(x2d, Wdkv, Wuk, Wuv, Wq, Wqr, Wkr, Wo)
    return out.reshape(1, S, D)


# device time: 108112 ns/iter; 1.7759x vs baseline; 1.2380x over previous
import jax
import jax.numpy as jnp
from jax import lax
from jax.experimental import pallas as pl
from jax.experimental.pallas import tpu as pltpu

S = 1024
D = 2048
DC = 128
H = 16
G = 4
DH = 128
DR = 32
SCALE = (DH + DR) ** -0.5

F32 = jnp.float32
BF16 = jnp.bfloat16


def kernel(x, Wdkv, Wuk, Wuv, Wq, Wqr, Wkr, Wo):
    x2d = x.reshape(S, D)

    def body(x_ref, wdkv_ref, wuk_ref, wuv_ref, wq_ref, wqr_ref, wkr_ref,
             wo_ref, out_ref,
             c_send, c_recv, wuk_send, wuk_recv, wuv_send, wuv_recv, kr_buf,
             send_sems, recv_sems):
        h = pl.program_id(0)
        my_x = lax.axis_index("x")
        my_y = lax.axis_index("y")
        my_z = lax.axis_index("z")
        peer = (1 - my_x, my_y, my_z)

        def mk_rdma(i, src, dst):
            return pltpu.make_async_remote_copy(
                src_ref=src, dst_ref=dst,
                send_sem=send_sems.at[i], recv_sem=recv_sems.at[i],
                device_id=peer, device_id_type=pl.DeviceIdType.MESH)

        @pl.when(h == 0)
        def _exchange():
            barrier_sem = pltpu.get_barrier_semaphore()
            pl.semaphore_signal(barrier_sem, inc=1, device_id=peer,
                                device_id_type=pl.DeviceIdType.MESH)
            pl.semaphore_wait(barrier_sem, 1)

            wuk_send[...] = wuk_ref[...].astype(BF16)
            mk_rdma(1, wuk_send, wuk_recv).start()
            wuv_send[...] = wuv_ref[...].astype(BF16)
            mk_rdma(2, wuv_send, wuv_recv).start()

            c_send[...] = jnp.dot(
                x_ref[...].astype(BF16), wdkv_ref[...].astype(BF16),
                preferred_element_type=F32).astype(BF16)
            mk_rdma(0, c_send, c_recv).start()

            kr_buf[...] = jnp.dot(x_ref[...], wkr_ref[...],
                                  preferred_element_type=F32)

        xb = x_ref[...].astype(BF16)
        q_blk = (jnp.dot(xb, wq_ref[...].astype(BF16),
                         preferred_element_type=F32)
                 * SCALE).astype(BF16)
        qr_blk = (jnp.dot(xb, wqr_ref[...].astype(BF16),
                          preferred_element_type=F32)
                  * SCALE).astype(BF16)

        @pl.when(h == 0)
        def _wait_exchange():
            mk_rdma(0, c_send, c_recv).wait()
            mk_rdma(1, wuk_send, wuk_recv).wait()
            mk_rdma(2, wuv_send, wuv_recv).wait()

        c_m = c_send[...]
        c_p = c_recv[...]
        kr = kr_buf[...].astype(BF16)

        blk = pl.ds(h * (G * DH), G * DH)
        k_blk = (jnp.dot(c_m, wuk_send[:, blk], preferred_element_type=F32)
                 + jnp.dot(c_p, wuk_recv[:, blk],
                           preferred_element_type=F32)).astype(BF16)
        v_blk = (jnp.dot(c_m, wuv_send[:, blk], preferred_element_type=F32)
                 + jnp.dot(c_p, wuv_recv[:, blk], preferred_element_type=F32))

        o_parts = []
        for j in range(G):
            k_h = k_blk[:, j * DH:(j + 1) * DH]
            v_h = v_blk[:, j * DH:(j + 1) * DH]
            q_h = q_blk[:, j * DH:(j + 1) * DH]
            qr_h = qr_blk[:, j * DR:(j + 1) * DR]
            s = lax.dot_general(q_h, k_h, (((1,), (1,)), ((), ())),
                                preferred_element_type=F32)
            s = s + lax.dot_general(qr_h, kr, (((1,), (1,)), ((), ())),
                                    preferred_element_type=F32)
            m = jnp.max(s, axis=1, keepdims=True)
            p = jnp.exp(s - m)
            r = 1.0 / jnp.sum(p, axis=1, keepdims=True)
            o_parts.append(jnp.dot(p, v_h, preferred_element_type=F32) * r)

        o_blk = jnp.concatenate(o_parts, axis=1).astype(BF16)
        contrib = jnp.dot(o_blk, wo_ref[...].astype(BF16),
                          preferred_element_type=F32)

        @pl.when(h == 0)
        def _init():
            out_ref[...] = contrib

        @pl.when(h != 0)
        def _acc():
            out_ref[...] = out_ref[...] + contrib

    out = pl.pallas_call(
        body,
        grid=(H // G,),
        out_shape=jax.ShapeDtypeStruct((S, D), F32),
        in_specs=[
            pl.BlockSpec((S, D), lambda h: (0, 0)),
            pl.BlockSpec((D, DC), lambda h: (0, 0)),
            pl.BlockSpec((DC, D), lambda h: (0, 0)),
            pl.BlockSpec((DC, D), lambda h: (0, 0)),
            pl.BlockSpec((D, G * DH), lambda h: (0, h)),
            pl.BlockSpec((D, G * DR), lambda h: (0, h)),
            pl.BlockSpec((D, DR), lambda h: (0, 0)),
            pl.BlockSpec((G * DH, D), lambda h: (h, 0)),
        ],
        out_specs=pl.BlockSpec((S, D), lambda h: (0, 0)),
        scratch_shapes=[
            pltpu.VMEM((S, DC), BF16),
            pltpu.VMEM((S, DC), BF16),
            pltpu.VMEM((DC, D), BF16),
            pltpu.VMEM((DC, D), BF16),
            pltpu.VMEM((DC, D), BF16),
            pltpu.VMEM((DC, D), BF16),
            pltpu.VMEM((S, DR), F32),
            pltpu.SemaphoreType.DMA((3,)),
            pltpu.SemaphoreType.DMA((3,)),
        ],
        compiler_params=pltpu.CompilerParams(
            collective_id=0,
            vmem_limit_bytes=64 * 1024 * 1024,
        ),
    )(x2d, Wdkv, Wuk, Wuv, Wq, Wqr, Wkr, Wo)
    return out.reshape(1, S, D)


# device time: 102306 ns/iter; 1.8767x vs baseline; 1.0568x over previous
import jax
import jax.numpy as jnp
from jax import lax
from jax.experimental import pallas as pl
from jax.experimental.pallas import tpu as pltpu

S = 1024
D = 2048
DC = 128
H = 16
G = 4
DH = 128
DR = 32
SCALE = (DH + DR) ** -0.5
LOG2E = 1.4426950408889634
QSCALE = SCALE * LOG2E

F32 = jnp.float32
BF16 = jnp.bfloat16


def kernel(x, Wdkv, Wuk, Wuv, Wq, Wqr, Wkr, Wo):
    x2d = x.reshape(S, D)

    def body(x_ref, wdkv_ref, wuk_ref, wuv_ref, wq_ref, wqr_ref, wkr_ref,
             wo_ref, out_ref,
             c_send, c_recv, wuk_send, wuk_recv, wuv_send, wuv_recv, kr_buf,
             send_sems, recv_sems):
        h = pl.program_id(0)
        my_x = lax.axis_index("x")
        my_y = lax.axis_index("y")
        my_z = lax.axis_index("z")
        peer = (1 - my_x, my_y, my_z)

        def mk_rdma(i, src, dst):
            return pltpu.make_async_remote_copy(
                src_ref=src, dst_ref=dst,
                send_sem=send_sems.at[i], recv_sem=recv_sems.at[i],
                device_id=peer, device_id_type=pl.DeviceIdType.MESH)

        @pl.when(h == 0)
        def _exchange():
            barrier_sem = pltpu.get_barrier_semaphore()
            pl.semaphore_signal(barrier_sem, inc=1, device_id=peer,
                                device_id_type=pl.DeviceIdType.MESH)
            pl.semaphore_wait(barrier_sem, 1)

            wuk_send[...] = wuk_ref[...].astype(BF16)
            mk_rdma(1, wuk_send, wuk_recv).start()
            wuv_send[...] = wuv_ref[...].astype(BF16)
            mk_rdma(2, wuv_send, wuv_recv).start()

            c_send[...] = jnp.dot(
                x_ref[...].astype(BF16), wdkv_ref[...].astype(BF16),
                preferred_element_type=F32).astype(BF16)
            mk_rdma(0, c_send, c_recv).start()

            kr_buf[...] = jnp.dot(x_ref[...], wkr_ref[...],
                                  preferred_element_type=F32)

        xb = x_ref[...].astype(BF16)
        q_blk = (jnp.dot(xb, wq_ref[...].astype(BF16),
                         preferred_element_type=F32)
                 * QSCALE).astype(BF16)
        qr_blk = (jnp.dot(xb, wqr_ref[...].astype(BF16),
                          preferred_element_type=F32)
                  * QSCALE).astype(BF16)

        @pl.when(h == 0)
        def _wait_exchange():
            mk_rdma(0, c_send, c_recv).wait()
            mk_rdma(1, wuk_send, wuk_recv).wait()
            mk_rdma(2, wuv_send, wuv_recv).wait()

        c_m = c_send[...]
        c_p = c_recv[...]
        kr = kr_buf[...].astype(BF16)

        blk = pl.ds(h * (G * DH), G * DH)
        k_blk = (jnp.dot(c_m, wuk_send[:, blk], preferred_element_type=F32)
                 + jnp.dot(c_p, wuk_recv[:, blk],
                           preferred_element_type=F32)).astype(BF16)
        v_blk = (jnp.dot(c_m, wuv_send[:, blk], preferred_element_type=F32)
                 + jnp.dot(c_p, wuv_recv[:, blk],
                           preferred_element_type=F32)).astype(BF16)

        o_parts = []
        for j in range(G):
            k_h = k_blk[:, j * DH:(j + 1) * DH]
            v_h = v_blk[:, j * DH:(j + 1) * DH]
            q_h = q_blk[:, j * DH:(j + 1) * DH]
            qr_h = qr_blk[:, j * DR:(j + 1) * DR]
            s = lax.dot_general(q_h, k_h, (((1,), (1,)), ((), ())),
                                preferred_element_type=F32)
            s = s + lax.dot_general(qr_h, kr, (((1,), (1,)), ((), ())),
                                    preferred_element_type=F32)
            p = jnp.exp2(s).astype(BF16)
            r = 1.0 / jnp.sum(p.astype(F32), axis=1, keepdims=True)
            o_parts.append(jnp.dot(p, v_h, preferred_element_type=F32) * r)

        o_blk = jnp.concatenate(o_parts, axis=1).astype(BF16)
        contrib = jnp.dot(o_blk, wo_ref[...].astype(BF16),
                          preferred_element_type=F32)

        @pl.when(h == 0)
        def _init():
            out_ref[...] = contrib

        @pl.when(h != 0)
        def _acc():
            out_ref[...] = out_ref[...] + contrib

    out = pl.pallas_call(
        body,
        grid=(H // G,),
        out_shape=jax.ShapeDtypeStruct((S, D), F32),
        in_specs=[
            pl.BlockSpec((S, D), lambda h: (0, 0)),
            pl.BlockSpec((D, DC), lambda h: (0, 0)),
            pl.BlockSpec((DC, D), lambda h: (0, 0)),
            pl.BlockSpec((DC, D), lambda h: (0, 0)),
            pl.BlockSpec((D, G * DH), lambda h: (0, h)),
            pl.BlockSpec((D, G * DR), lambda h: (0, h)),
            pl.BlockSpec((D, DR), lambda h: (0, 0)),
            pl.BlockSpec((G * DH, D), lambda h: (h, 0)),
        ],
        out_specs=pl.BlockSpec((S, D), lambda h: (0, 0)),
        scratch_shapes=[
            pltpu.VMEM((S, DC), BF16),
            pltpu.VMEM((S, DC), BF16),
            pltpu.VMEM((DC, D), BF16),
            pltpu.VMEM((DC, D), BF16),
            pltpu.VMEM((DC, D), BF16),
            pltpu.VMEM((DC, D), BF16),
            pltpu.VMEM((S, DR), F32),
            pltpu.SemaphoreType.DMA((3,)),
            pltpu.SemaphoreType.DMA((3,)),
        ],
        compiler_params=pltpu.CompilerParams(
            collective_id=0,
            vmem_limit_bytes=64 * 1024 * 1024,
        ),
    )(x2d, Wdkv, Wuk, Wuv, Wq, Wqr, Wkr, Wo)
    return out.reshape(1, S, D)


# device time: 97106 ns/iter; 1.9772x vs baseline; 1.0535x over previous
import jax
import jax.numpy as jnp
from jax import lax
from jax.experimental import pallas as pl
from jax.experimental.pallas import tpu as pltpu

S = 1024
D = 2048
DC = 128
H = 16
G = 4
DH = 128
DR = 32
NSTEP = H // G
SCALE = (DH + DR) ** -0.5
LOG2E = 1.4426950408889634
QSCALE = SCALE * LOG2E

F32 = jnp.float32
BF16 = jnp.bfloat16


def kernel(x, Wdkv, Wuk, Wuv, Wq, Wqr, Wkr, Wo):
    x2d = x.reshape(S, D)

    def body(x_ref, wdkv_ref, wuk_ref, wuv_ref, wq_ref, wqr_ref, wkr_ref,
             wo_ref, out_ref,
             c_send, c_recv, wuk_send, wuk_recv, wuv_send, wuv_recv, kr_buf,
             send_sems, recv_sems):
        h = pl.program_id(0)
        my_x = lax.axis_index("x")
        my_y = lax.axis_index("y")
        my_z = lax.axis_index("z")
        peer = (1 - my_x, my_y, my_z)

        CW = G * DH

        def mk_rdma(i, src, dst):
            return pltpu.make_async_remote_copy(
                src_ref=src, dst_ref=dst,
                send_sem=send_sems.at[i], recv_sem=recv_sems.at[i],
                device_id=peer, device_id_type=pl.DeviceIdType.MESH)

        def wuk_rdma(g):
            sl = slice(g * CW, (g + 1) * CW)
            return mk_rdma(1 + g, wuk_send.at[:, sl], wuk_recv.at[:, sl])

        def wuv_rdma(g):
            sl = slice(g * CW, (g + 1) * CW)
            return mk_rdma(1 + NSTEP + g, wuv_send.at[:, sl],
                           wuv_recv.at[:, sl])

        @pl.when(h == 0)
        def _exchange():
            barrier_sem = pltpu.get_barrier_semaphore()
            pl.semaphore_signal(barrier_sem, inc=1, device_id=peer,
                                device_id_type=pl.DeviceIdType.MESH)
            pl.semaphore_wait(barrier_sem, 1)

            c_send[...] = jnp.dot(
                x_ref[...].astype(BF16), wdkv_ref[...].astype(BF16),
                preferred_element_type=F32).astype(BF16)
            mk_rdma(0, c_send, c_recv).start()

            wuk_send[...] = wuk_ref[...].astype(BF16)
            wuv_send[...] = wuv_ref[...].astype(BF16)
            for g in range(NSTEP):
                wuk_rdma(g).start()
                wuv_rdma(g).start()

            kr_buf[...] = jnp.dot(x_ref[...], wkr_ref[...],
                                  preferred_element_type=F32)

        xb = x_ref[...].astype(BF16)
        q_blk = (jnp.dot(xb, wq_ref[...].astype(BF16),
                         preferred_element_type=F32)
                 * QSCALE).astype(BF16)
        qr_blk = (jnp.dot(xb, wqr_ref[...].astype(BF16),
                          preferred_element_type=F32)
                  * QSCALE).astype(BF16)

        @pl.when(h == 0)
        def _wait_c():
            mk_rdma(0, c_send, c_recv).wait()

        for g in range(NSTEP):
            @pl.when(h == g)
            def _wait_chunks(g=g):
                wuk_rdma(g).wait()
                wuv_rdma(g).wait()

        c_m = c_send[...]
        c_p = c_recv[...]
        kr = kr_buf[...].astype(BF16)

        blk = pl.ds(h * (G * DH), G * DH)
        k_blk = (jnp.dot(c_m, wuk_send[:, blk], preferred_element_type=F32)
                 + jnp.dot(c_p, wuk_recv[:, blk],
                           preferred_element_type=F32)).astype(BF16)
        v_blk = (jnp.dot(c_m, wuv_send[:, blk], preferred_element_type=F32)
                 + jnp.dot(c_p, wuv_recv[:, blk],
                           preferred_element_type=F32)).astype(BF16)

        o_parts = []
        for j in range(G):
            k_h = k_blk[:, j * DH:(j + 1) * DH]
            v_h = v_blk[:, j * DH:(j + 1) * DH]
            q_h = q_blk[:, j * DH:(j + 1) * DH]
            qr_h = qr_blk[:, j * DR:(j + 1) * DR]
            s = lax.dot_general(q_h, k_h, (((1,), (1,)), ((), ())),
                                preferred_element_type=F32)
            s = s + lax.dot_general(qr_h, kr, (((1,), (1,)), ((), ())),
                                    preferred_element_type=F32)
            p = jnp.exp2(s).astype(BF16)
            r = 1.0 / jnp.sum(p.astype(F32), axis=1, keepdims=True)
            o_parts.append(jnp.dot(p, v_h, preferred_element_type=F32) * r)

        o_blk = jnp.concatenate(o_parts, axis=1).astype(BF16)
        contrib = jnp.dot(o_blk, wo_ref[...].astype(BF16),
                          preferred_element_type=F32)

        @pl.when(h == 0)
        def _init():
            out_ref[...] = contrib

        @pl.when(h != 0)
        def _acc():
            out_ref[...] = out_ref[...] + contrib

    out = pl.pallas_call(
        body,
        grid=(H // G,),
        out_shape=jax.ShapeDtypeStruct((S, D), F32),
        in_specs=[
            pl.BlockSpec((S, D), lambda h: (0, 0)),
            pl.BlockSpec((D, DC), lambda h: (0, 0)),
            pl.BlockSpec((DC, D), lambda h: (0, 0)),
            pl.BlockSpec((DC, D), lambda h: (0, 0)),
            pl.BlockSpec((D, G * DH), lambda h: (0, h)),
            pl.BlockSpec((D, G * DR), lambda h: (0, h)),
            pl.BlockSpec((D, DR), lambda h: (0, 0)),
            pl.BlockSpec((G * DH, D), lambda h: (h, 0)),
        ],
        out_specs=pl.BlockSpec((S, D), lambda h: (0, 0)),
        scratch_shapes=[
            pltpu.VMEM((S, DC), BF16),
            pltpu.VMEM((S, DC), BF16),
            pltpu.VMEM((DC, D), BF16),
            pltpu.VMEM((DC, D), BF16),
            pltpu.VMEM((DC, D), BF16),
            pltpu.VMEM((DC, D), BF16),
            pltpu.VMEM((S, DR), F32),
            pltpu.SemaphoreType.DMA((1 + 2 * NSTEP,)),
            pltpu.SemaphoreType.DMA((1 + 2 * NSTEP,)),
        ],
        compiler_params=pltpu.CompilerParams(
            collective_id=0,
            vmem_limit_bytes=64 * 1024 * 1024,
        ),
    )(x2d, Wdkv, Wuk, Wuv, Wq, Wqr, Wkr, Wo)
    return out.reshape(1, S, D)


# device time: 96934 ns/iter; 1.9807x vs baseline; 1.0018x over previous
import jax
import jax.numpy as jnp
from jax import lax
from jax.experimental import pallas as pl
from jax.experimental.pallas import tpu as pltpu

S = 1024
D = 2048
DC = 128
H = 16
G = 4
DH = 128
DR = 32
NSTEP = H // G
SCALE = (DH + DR) ** -0.5
LOG2E = 1.4426950408889634
QSCALE = SCALE * LOG2E

F32 = jnp.float32
BF16 = jnp.bfloat16


def kernel(x, Wdkv, Wuk, Wuv, Wq, Wqr, Wkr, Wo):
    def body(x_ref, wdkv_ref, wuk_ref, wuv_ref, wq_ref, wqr_ref, wkr_ref,
             wo_ref, out_ref,
             c_send, c_recv, wuk_send, wuk_recv, wuv_send, wuv_recv, kr_buf,
             send_sems, recv_sems):
        h = pl.program_id(0)
        my_x = lax.axis_index("x")
        my_y = lax.axis_index("y")
        my_z = lax.axis_index("z")
        peer = (1 - my_x, my_y, my_z)

        CW = G * DH

        def mk_rdma(i, src, dst):
            return pltpu.make_async_remote_copy(
                src_ref=src, dst_ref=dst,
                send_sem=send_sems.at[i], recv_sem=recv_sems.at[i],
                device_id=peer, device_id_type=pl.DeviceIdType.MESH)

        def wuk_rdma(g):
            sl = slice(g * CW, (g + 1) * CW)
            return mk_rdma(1 + g, wuk_send.at[:, sl], wuk_recv.at[:, sl])

        def wuv_rdma(g):
            sl = slice(g * CW, (g + 1) * CW)
            return mk_rdma(1 + NSTEP + g, wuv_send.at[:, sl],
                           wuv_recv.at[:, sl])

        @pl.when(h == 0)
        def _exchange():
            barrier_sem = pltpu.get_barrier_semaphore()
            pl.semaphore_signal(barrier_sem, inc=1, device_id=peer,
                                device_id_type=pl.DeviceIdType.MESH)
            pl.semaphore_wait(barrier_sem, 1)

            c_send[...] = jnp.dot(
                x_ref[0].astype(BF16), wdkv_ref[...].astype(BF16),
                preferred_element_type=F32).astype(BF16)
            mk_rdma(0, c_send, c_recv).start()

            wuk_send[...] = wuk_ref[...].astype(BF16)
            wuv_send[...] = wuv_ref[...].astype(BF16)
            for g in range(NSTEP):
                wuk_rdma(g).start()
                wuv_rdma(g).start()

            kr_buf[...] = jnp.dot(x_ref[0], wkr_ref[...],
                                  preferred_element_type=F32)

        xb = x_ref[0].astype(BF16)
        q_blk = (jnp.dot(xb, wq_ref[...].astype(BF16),
                         preferred_element_type=F32)
                 * QSCALE).astype(BF16)
        qr_blk = (jnp.dot(xb, wqr_ref[...].astype(BF16),
                          preferred_element_type=F32)
                  * QSCALE).astype(BF16)

        @pl.when(h == 0)
        def _wait_c():
            mk_rdma(0, c_send, c_recv).wait_recv()

        for g in range(NSTEP):
            @pl.when(h == g)
            def _wait_chunks(g=g):
                wuk_rdma(g).wait_recv()
                wuv_rdma(g).wait_recv()

        @pl.when(h == NSTEP - 1)
        def _wait_sends():
            mk_rdma(0, c_send, c_recv).wait_send()
            for g in range(NSTEP):
                wuk_rdma(g).wait_send()
                wuv_rdma(g).wait_send()

        c_m = c_send[...]
        c_p = c_recv[...]
        kr = kr_buf[...].astype(BF16)

        blk = pl.ds(h * (G * DH), G * DH)
        k_blk = (jnp.dot(c_m, wuk_send[:, blk], preferred_element_type=F32)
                 + jnp.dot(c_p, wuk_recv[:, blk],
                           preferred_element_type=F32)).astype(BF16)
        v_blk = (jnp.dot(c_m, wuv_send[:, blk], preferred_element_type=F32)
                 + jnp.dot(c_p, wuv_recv[:, blk],
                           preferred_element_type=F32)).astype(BF16)

        o_parts = []
        for j in range(G):
            k_h = k_blk[:, j * DH:(j + 1) * DH]
            v_h = v_blk[:, j * DH:(j + 1) * DH]
            q_h = q_blk[:, j * DH:(j + 1) * DH]
            qr_h = qr_blk[:, j * DR:(j + 1) * DR]
            s = lax.dot_general(q_h, k_h, (((1,), (1,)), ((), ())),
                                preferred_element_type=F32)
            s = s + lax.dot_general(qr_h, kr, (((1,), (1,)), ((), ())),
                                    preferred_element_type=F32)
            p = jnp.exp2(s).astype(BF16)
            r = 1.0 / jnp.sum(p.astype(F32), axis=1, keepdims=True)
            o_parts.append(jnp.dot(p, v_h, preferred_element_type=F32) * r)

        o_blk = jnp.concatenate(o_parts, axis=1).astype(BF16)
        contrib = jnp.dot(o_blk, wo_ref[...].astype(BF16),
                          preferred_element_type=F32)

        @pl.when(h == 0)
        def _init():
            out_ref[0] = contrib

        @pl.when(h != 0)
        def _acc():
            out_ref[0] = out_ref[0] + contrib

    out = pl.pallas_call(
        body,
        grid=(H // G,),
        out_shape=jax.ShapeDtypeStruct((1, S, D), F32),
        in_specs=[
            pl.BlockSpec((1, S, D), lambda h: (0, 0, 0)),
            pl.BlockSpec((D, DC), lambda h: (0, 0)),
            pl.BlockSpec((DC, D), lambda h: (0, 0)),
            pl.BlockSpec((DC, D), lambda h: (0, 0)),
            pl.BlockSpec((D, G * DH), lambda h: (0, h)),
            pl.BlockSpec((D, G * DR), lambda h: (0, h)),
            pl.BlockSpec((D, DR), lambda h: (0, 0)),
            pl.BlockSpec((G * DH, D), lambda h: (h, 0)),
        ],
        out_specs=pl.BlockSpec((1, S, D), lambda h: (0, 0, 0)),
        scratch_shapes=[
            pltpu.VMEM((S, DC), BF16),
            pltpu.VMEM((S, DC), BF16),
            pltpu.VMEM((DC, D), BF16),
            pltpu.VMEM((DC, D), BF16),
            pltpu.VMEM((DC, D), BF16),
            pltpu.VMEM((DC, D), BF16),
            pltpu.VMEM((S, DR), F32),
            pltpu.SemaphoreType.DMA((1 + 2 * NSTEP,)),
            pltpu.SemaphoreType.DMA((1 + 2 * NSTEP,)),
        ],
        compiler_params=pltpu.CompilerParams(
            collective_id=0,
            vmem_limit_bytes=64 * 1024 * 1024,
        ),
    )(x, Wdkv, Wuk, Wuv, Wq, Wqr, Wkr, Wo)
    return out


# device time: 94408 ns/iter; 2.0337x vs baseline; 1.0268x over previous
import jax
import jax.numpy as jnp
from jax import lax
from jax.experimental import pallas as pl
from jax.experimental.pallas import tpu as pltpu

S = 1024
D = 2048
DC = 128
H = 16
G = 4
DH = 128
DR = 32
NSTEP = H // G
SCALE = (DH + DR) ** -0.5
LOG2E = 1.4426950408889634
QSCALE = SCALE * LOG2E

F32 = jnp.float32
BF16 = jnp.bfloat16


def kernel(x, Wdkv, Wuk, Wuv, Wq, Wqr, Wkr, Wo):
    def body(x_ref, wdkv_ref, wuk_ref, wuv_ref, wq_ref, wqr_ref, wkr_ref,
             wo_ref, out_ref,
             c_send, c_recv, wuk_send, wuk_recv, wuv_send, wuv_recv, kr_buf,
             send_sems, recv_sems):
        h = pl.program_id(0)
        my_x = lax.axis_index("x")
        my_y = lax.axis_index("y")
        my_z = lax.axis_index("z")
        peer = (1 - my_x, my_y, my_z)

        CW = G * DH

        def mk_rdma(i, src, dst):
            return pltpu.make_async_remote_copy(
                src_ref=src, dst_ref=dst,
                send_sem=send_sems.at[i], recv_sem=recv_sems.at[i],
                device_id=peer, device_id_type=pl.DeviceIdType.MESH)

        def wuk_rdma(g):
            sl = slice(g * CW, (g + 1) * CW)
            return mk_rdma(1 + g, wuk_send.at[:, sl], wuk_recv.at[:, sl])

        def wuv_rdma(g):
            sl = slice(g * CW, (g + 1) * CW)
            return mk_rdma(1 + NSTEP + g, wuv_send.at[:, sl],
                           wuv_recv.at[:, sl])

        @pl.when(h == 0)
        def _exchange():
            barrier_sem = pltpu.get_barrier_semaphore()
            pl.semaphore_signal(barrier_sem, inc=1, device_id=peer,
                                device_id_type=pl.DeviceIdType.MESH)
            pl.semaphore_wait(barrier_sem, 1)

            c_send[...] = jnp.dot(
                x_ref[0].astype(BF16), wdkv_ref[...].astype(BF16),
                preferred_element_type=F32).astype(BF16)
            mk_rdma(0, c_send, c_recv).start()

            c0 = slice(0, CW)
            wuk_send[:, c0] = wuk_ref[:, c0].astype(BF16)
            wuv_send[:, c0] = wuv_ref[:, c0].astype(BF16)
            wuk_rdma(0).start()
            wuv_rdma(0).start()
            rest = slice(CW, D)
            wuk_send[:, rest] = wuk_ref[:, rest].astype(BF16)
            wuv_send[:, rest] = wuv_ref[:, rest].astype(BF16)
            for g in range(1, NSTEP):
                wuk_rdma(g).start()
                wuv_rdma(g).start()

            kr_buf[...] = lax.dot_general(
                x_ref[0], wkr_ref[...], (((1,), (1,)), ((), ())),
                preferred_element_type=F32)

        xb = x_ref[0].astype(BF16)
        q_blk = (jnp.dot(xb, wq_ref[...].astype(BF16),
                         preferred_element_type=F32)
                 * QSCALE).astype(BF16)
        qr_blk = (jnp.dot(xb, wqr_ref[...].astype(BF16),
                          preferred_element_type=F32)
                  * QSCALE).astype(BF16)

        @pl.when(h == 0)
        def _wait_c():
            mk_rdma(0, c_send, c_recv).wait_recv()

        for g in range(NSTEP):
            @pl.when(h == g)
            def _wait_chunks(g=g):
                wuk_rdma(g).wait_recv()
                wuv_rdma(g).wait_recv()

        @pl.when(h == NSTEP - 1)
        def _wait_sends():
            mk_rdma(0, c_send, c_recv).wait_send()
            for g in range(NSTEP):
                wuk_rdma(g).wait_send()
                wuv_rdma(g).wait_send()

        c_m = c_send[...]
        c_p = c_recv[...]
        kr = kr_buf[...].astype(BF16)

        blk = pl.ds(h * (G * DH), G * DH)
        k_blk = (jnp.dot(c_m, wuk_send[:, blk], preferred_element_type=F32)
                 + jnp.dot(c_p, wuk_recv[:, blk],
                           preferred_element_type=F32)).astype(BF16)
        v_blk = (jnp.dot(c_m, wuv_send[:, blk], preferred_element_type=F32)
                 + jnp.dot(c_p, wuv_recv[:, blk],
                           preferred_element_type=F32)).astype(BF16)

        o_parts = []
        for j in range(G):
            k_h = k_blk[:, j * DH:(j + 1) * DH]
            v_h = v_blk[:, j * DH:(j + 1) * DH]
            q_h = q_blk[:, j * DH:(j + 1) * DH]
            qr_h = qr_blk[:, j * DR:(j + 1) * DR]
            s = lax.dot_general(q_h, k_h, (((1,), (1,)), ((), ())),
                                preferred_element_type=F32)
            s = s + lax.dot_general(qr_h, kr, (((1,), (1,)), ((), ())),
                                    preferred_element_type=F32)
            p = jnp.exp2(s).astype(BF16)
            r = 1.0 / jnp.sum(p.astype(F32), axis=1, keepdims=True)
            o_parts.append(jnp.dot(p, v_h, preferred_element_type=F32) * r)

        o_blk = jnp.concatenate(o_parts, axis=1).astype(BF16)
        contrib = jnp.dot(o_blk, wo_ref[...].astype(BF16),
                          preferred_element_type=F32)

        @pl.when(h == 0)
        def _init():
            out_ref[0] = contrib

        @pl.when(h != 0)
        def _acc():
            out_ref[0] = out_ref[0] + contrib

    out = pl.pallas_call(
        body,
        grid=(H // G,),
        out_shape=jax.ShapeDtypeStruct((1, S, D), F32),
        in_specs=[
            pl.BlockSpec((1, S, D), lambda h: (0, 0, 0)),
            pl.BlockSpec((D, DC), lambda h: (0, 0)),
            pl.BlockSpec((DC, D), lambda h: (0, 0)),
            pl.BlockSpec((DC, D), lambda h: (0, 0)),
            pl.BlockSpec((D, G * DH), lambda h: (0, h)),
            pl.BlockSpec((D, G * DR), lambda h: (0, h)),
            pl.BlockSpec((DR, D), lambda h: (0, 0)),
            pl.BlockSpec((G * DH, D), lambda h: (h, 0)),
        ],
        out_specs=pl.BlockSpec((1, S, D), lambda h: (0, 0, 0)),
        scratch_shapes=[
            pltpu.VMEM((S, DC), BF16),
            pltpu.VMEM((S, DC), BF16),
            pltpu.VMEM((DC, D), BF16),
            pltpu.VMEM((DC, D), BF16),
            pltpu.VMEM((DC, D), BF16),
            pltpu.VMEM((DC, D), BF16),
            pltpu.VMEM((S, DR), F32),
            pltpu.SemaphoreType.DMA((1 + 2 * NSTEP,)),
            pltpu.SemaphoreType.DMA((1 + 2 * NSTEP,)),
        ],
        compiler_params=pltpu.CompilerParams(
            collective_id=0,
            vmem_limit_bytes=64 * 1024 * 1024,
        ),
    )(x, Wdkv, Wuk, Wuv, Wq, Wqr, Wkr.T, Wo)
    return out
